# baseline (speedup 1.0000x reference)
import functools

import jax
import jax.numpy as jnp
from jax import lax
from jax.experimental import pallas as pl
from jax.experimental.pallas import tpu as pltpu
from jax.experimental.pallas import tpu_sc as plsc

B, S, D = 4, 8192, 768
EPS = 1e-12
L = 16
NW = 32
CH = 16
ROWS_PER_W = S // NW
NCHUNK = ROWS_PER_W // CH
NJ = D // L
INV_D = 1.0 / D


def _splat(x, dtype=jnp.float32):
    return jnp.full((L,), x, dtype=dtype)


def _dyn_gather(v, idx):
    return lax.gather(
        v, idx[:, None],
        dimension_numbers=lax.GatherDimensionNumbers(
            offset_dims=(), collapsed_slice_dims=(0,), start_index_map=(0,)),
        slice_sizes=(1,),
        mode=lax.GatherScatterMode.PROMISE_IN_BOUNDS)


def _lane_sum(v):
    iota = lax.iota(jnp.int32, L)
    for k in (1, 2, 4, 8):
        v = v + _dyn_gather(v, jnp.bitwise_xor(iota, k))
    return v


def _newton_rsqrt(v):
    vi = lax.bitcast_convert_type(v, jnp.int32)
    yi = _splat(0x5F3759DF, jnp.int32) - lax.shift_right_logical(vi, _splat(1, jnp.int32))
    y = lax.bitcast_convert_type(yi, jnp.float32)
    half_v = 0.5 * v
    for _ in range(3):
        y = y * (1.5 - half_v * y * y)
    return y


def _sc_body(concat_hbm, type_hbm, pos_hbm, tok_hbm, w_hbm, b_hbm, out_hbm,
             tok_v, tokd_v, w_v, b_v, pos_v, in_v, out_v, t_v):
    wid = lax.axis_index("s") * 2 + lax.axis_index("c")
    base = wid * ROWS_PER_W

    pltpu.sync_copy(tok_hbm, tok_v)
    pltpu.sync_copy(w_hbm, w_v)
    pltpu.sync_copy(b_hbm, b_v)
    for j in range(NJ):
        dj = pl.ds(j * L, L)
        tokd_v[dj] = tok_v[1, dj] - tok_v[0, dj]

    def chunk_body(c, _):
        s0 = base + c * CH
        g = wid * NCHUNK + c
        pltpu.sync_copy(pos_hbm.at[pl.ds(s0, CH)], pos_v)
        pltpu.sync_copy(type_hbm.at[g], t_v)

        def fold_row(r, _):
            for j in range(NJ):
                dj = pl.ds(j * L, L)
                pos_v[r, dj] = pos_v[r, dj] + tok_v[0, dj]
            return 0
        lax.fori_loop(0, CH, fold_row, 0)

        for b in range(B):
            pltpu.sync_copy(concat_hbm.at[b, pl.ds(s0, CH)], in_v)
            t_row = t_v[b, :]

            def row_body(r, _, t_row=t_row):
                t = _dyn_gather(t_row, jnp.full((L,), r, jnp.int32))
                acc = _splat(0.0)
                accsq = _splat(0.0)
                for j in range(NJ):
                    dj = pl.ds(j * L, L)
                    x = in_v[r, dj] + pos_v[r, dj] + t * tokd_v[dj]
                    in_v[r, dj] = x
                    acc = acc + x
                    accsq = accsq + x * x
                u_spl = _lane_sum(acc) * INV_D
                var_spl = _lane_sum(accsq) * INV_D - u_spl * u_spl
                y = _newton_rsqrt(var_spl + EPS)
                for j in range(NJ):
                    dj = pl.ds(j * L, L)
                    out_v[r, dj] = (in_v[r, dj] - u_spl) * y * w_v[dj] + b_v[dj]
                return 0

            lax.fori_loop(0, CH, row_body, 0)
            pltpu.sync_copy(out_v, out_hbm.at[b, pl.ds(s0, CH)])
        return 0

    lax.fori_loop(0, NCHUNK, chunk_body, 0)


@jax.jit
def kernel(concat_embeddings, concat_type, pos_emb, tok_emb, ln_weight, ln_bias):
    type_r = (concat_type.astype(jnp.float32)
              .reshape(B, S // CH, CH).transpose(1, 0, 2))
    mesh = plsc.VectorSubcoreMesh(core_axis_name="c", subcore_axis_name="s")
    run = functools.partial(
        pl.kernel,
        mesh=mesh,
        out_type=jax.ShapeDtypeStruct((B, S, D), jnp.float32),
        scratch_types=[
            pltpu.VMEM((2, D), jnp.float32),
            pltpu.VMEM((D,), jnp.float32),
            pltpu.VMEM((D,), jnp.float32),
            pltpu.VMEM((D,), jnp.float32),
            pltpu.VMEM((CH, D), jnp.float32),
            pltpu.VMEM((CH, D), jnp.float32),
            pltpu.VMEM((CH, D), jnp.float32),
            pltpu.VMEM((B, CH), jnp.float32),
        ],
    )(_sc_body)
    return run(concat_embeddings, type_r, pos_emb, tok_emb, ln_weight, ln_bias)
